# baseline (speedup 1.0000x reference)
import jax
import jax.numpy as jnp
from jax.experimental import pallas as pl
from jax.experimental.pallas import tpu as pltpu


def _lstm_gates(gates, c, H):
    sig = jax.nn.sigmoid(gates[:, :3 * H])
    i_g = sig[:, 0:H]
    f_g = sig[:, H:2 * H]
    o_g = sig[:, 2 * H:3 * H]
    g_g = jnp.tanh(gates[:, 3 * H:])
    c_new = f_g * c + i_g * g_g
    h_new = o_g * jnp.tanh(c_new)
    return h_new, c_new


def _scan_chunk(gxf_ref, gxb_ref, whh_f, whh_b, outf_ref, outb_ref,
                hf_ref, cf_ref, hb_ref, cb_ref, Tc, Bp, H):
    def body(j, carry):
        h_f, c_f, h_b, c_b = carry
        rf = pl.multiple_of(j * Bp, Bp)
        rb = pl.multiple_of((Tc - 1 - j) * Bp, Bp)
        gf = (gxf_ref[pl.ds(rf, Bp), :].astype(jnp.float32)
              + jnp.dot(h_f, whh_f, preferred_element_type=jnp.float32))
        gb = (gxb_ref[pl.ds(rb, Bp), :].astype(jnp.float32)
              + jnp.dot(h_b, whh_b, preferred_element_type=jnp.float32))
        h_f, c_f = _lstm_gates(gf, c_f, H)
        h_b, c_b = _lstm_gates(gb, c_b, H)
        outf_ref[j] = h_f.astype(outf_ref.dtype)
        outb_ref[Tc - 1 - j] = h_b.astype(outb_ref.dtype)
        return h_f, c_f, h_b, c_b

    carry = (hf_ref[...], cf_ref[...], hb_ref[...], cb_ref[...])
    carry = jax.lax.fori_loop(0, Tc, body, carry, unroll=Tc)
    h_f, c_f, h_b, c_b = carry
    hf_ref[...] = h_f
    cf_ref[...] = c_f
    hb_ref[...] = h_b
    cb_ref[...] = c_b


def _layer0_kernel(xf_ref, xb_ref, wih_ref, whh_ref, b_ref,
                   outf_ref, outb_ref,
                   hf_ref, cf_ref, hb_ref, cb_ref,
                   gxfA_ref, gxbA_ref, gxfB_ref, gxbB_ref):
    k = pl.program_id(0)

    @pl.when(k <= 1)
    def _():
        hf_ref[...] = jnp.zeros_like(hf_ref)
        cf_ref[...] = jnp.zeros_like(cf_ref)
        hb_ref[...] = jnp.zeros_like(hb_ref)
        cb_ref[...] = jnp.zeros_like(cb_ref)

    Bp, Tc, D = xf_ref.shape
    H = hf_ref.shape[1]
    R = Tc * Bp

    def gx_and_scan(gxw_f, gxw_b, gxr_f, gxr_b):
        xf2 = jnp.swapaxes(xf_ref[...], 0, 1).reshape(R, D)
        xb2 = jnp.swapaxes(xb_ref[...], 0, 1).reshape(R, D)
        gxw_f[...] = (
            jnp.dot(xf2, wih_ref[0], preferred_element_type=jnp.float32)
            + b_ref[0]).astype(gxw_f.dtype)
        gxw_b[...] = (
            jnp.dot(xb2, wih_ref[1], preferred_element_type=jnp.float32)
            + b_ref[1]).astype(gxw_b.dtype)
        _scan_chunk(gxr_f, gxr_b, whh_ref[0], whh_ref[1],
                    outf_ref, outb_ref, hf_ref, cf_ref, hb_ref, cb_ref,
                    Tc, Bp, H)

    @pl.when(k % 2 == 0)
    def _():
        gx_and_scan(gxfA_ref, gxbA_ref, gxfB_ref, gxbB_ref)

    @pl.when(k % 2 == 1)
    def _():
        gx_and_scan(gxfB_ref, gxbB_ref, gxfA_ref, gxbA_ref)


def _layer1_kernel(ff_ref, fb_ref, bf_ref, bb_ref, wih_ref, whh_ref, b_ref,
                   outf_ref, outb_ref,
                   hf_ref, cf_ref, hb_ref, cb_ref,
                   gxfA_ref, gxbA_ref, gxfB_ref, gxbB_ref,
                   sf_ref, sb_ref):
    k = pl.program_id(0)

    @pl.when(k <= 1)
    def _():
        hf_ref[...] = jnp.zeros_like(hf_ref)
        cf_ref[...] = jnp.zeros_like(cf_ref)
        hb_ref[...] = jnp.zeros_like(hb_ref)
        cb_ref[...] = jnp.zeros_like(cb_ref)

    Tc, Bp, H = ff_ref.shape
    R = Tc * Bp
    wih_f = wih_ref[0]
    wih_b = wih_ref[1]

    def gx_and_scan(gxw_f, gxw_b, gxr_f, gxr_b):
        gxw_f[...] = (
            jnp.dot(ff_ref[...].reshape(R, H), wih_f[:H],
                    preferred_element_type=jnp.float32)
            + jnp.dot(bf_ref[...].reshape(R, H), wih_f[H:],
                      preferred_element_type=jnp.float32)
            + b_ref[0]).astype(gxw_f.dtype)
        gxw_b[...] = (
            jnp.dot(fb_ref[...].reshape(R, H), wih_b[:H],
                    preferred_element_type=jnp.float32)
            + jnp.dot(bb_ref[...].reshape(R, H), wih_b[H:],
                      preferred_element_type=jnp.float32)
            + b_ref[1]).astype(gxw_b.dtype)
        _scan_chunk(gxr_f, gxr_b, whh_ref[0], whh_ref[1],
                    sf_ref, sb_ref, hf_ref, cf_ref, hb_ref, cb_ref,
                    Tc, Bp, H)
        outf_ref[...] = jnp.swapaxes(sf_ref[...], 0, 1)
        outb_ref[...] = jnp.swapaxes(sb_ref[...], 0, 1)

    @pl.when(k % 2 == 0)
    def _():
        gx_and_scan(gxfA_ref, gxbA_ref, gxfB_ref, gxbB_ref)

    @pl.when(k % 2 == 1)
    def _():
        gx_and_scan(gxfB_ref, gxbB_ref, gxfA_ref, gxbA_ref)


_VMEM_LIMIT = 100 * 1024 * 1024


def _run_layer0(x_bt, wih, whh, bias, T, Bp, H, Tc):
    D = x_bt.shape[-1]
    nchunks = T // Tc
    n1 = nchunks - 1

    def fwd_in(k):
        return (0, jnp.minimum(k, n1), 0)

    def bwd_in(k):
        return (0, n1 - jnp.minimum(k, n1), 0)

    def fwd_out(k):
        return (jnp.maximum(k - 1, 0), 0, 0)

    def bwd_out(k):
        return (n1 - jnp.maximum(k - 1, 0), 0, 0)

    return pl.pallas_call(
        _layer0_kernel,
        out_shape=(jax.ShapeDtypeStruct((T, Bp, H), jnp.bfloat16),
                   jax.ShapeDtypeStruct((T, Bp, H), jnp.bfloat16)),
        grid_spec=pltpu.PrefetchScalarGridSpec(
            num_scalar_prefetch=0,
            grid=(nchunks + 1,),
            in_specs=[
                pl.BlockSpec((Bp, Tc, D), fwd_in),
                pl.BlockSpec((Bp, Tc, D), bwd_in),
                pl.BlockSpec((2, D, 4 * H), lambda k: (0, 0, 0),
                             pipeline_mode=pl.Buffered(1)),
                pl.BlockSpec((2, H, 4 * H), lambda k: (0, 0, 0),
                             pipeline_mode=pl.Buffered(1)),
                pl.BlockSpec((2, 1, 4 * H), lambda k: (0, 0, 0),
                             pipeline_mode=pl.Buffered(1)),
            ],
            out_specs=(
                pl.BlockSpec((Tc, Bp, H), fwd_out),
                pl.BlockSpec((Tc, Bp, H), bwd_out),
            ),
            scratch_shapes=[
                pltpu.VMEM((Bp, H), jnp.float32),
                pltpu.VMEM((Bp, H), jnp.float32),
                pltpu.VMEM((Bp, H), jnp.float32),
                pltpu.VMEM((Bp, H), jnp.float32),
                pltpu.VMEM((Tc * Bp, 4 * H), jnp.bfloat16),
                pltpu.VMEM((Tc * Bp, 4 * H), jnp.bfloat16),
                pltpu.VMEM((Tc * Bp, 4 * H), jnp.bfloat16),
                pltpu.VMEM((Tc * Bp, 4 * H), jnp.bfloat16),
            ],
        ),
        compiler_params=pltpu.CompilerParams(
            dimension_semantics=("arbitrary",),
            vmem_limit_bytes=_VMEM_LIMIT),
    )(x_bt, x_bt, wih, whh, bias)


def _run_layer1(outf, outb, wih, whh, bias, T, Bp, H, Tc):
    nchunks = T // Tc
    n1 = nchunks - 1

    def fwd_in(k):
        return (jnp.minimum(k, n1), 0, 0)

    def bwd_in(k):
        return (n1 - jnp.minimum(k, n1), 0, 0)

    def fwd_out(k):
        return (0, jnp.maximum(k - 1, 0), 0)

    def bwd_out(k):
        return (0, n1 - jnp.maximum(k - 1, 0), 0)

    return pl.pallas_call(
        _layer1_kernel,
        out_shape=(jax.ShapeDtypeStruct((Bp, T, H), jnp.float32),
                   jax.ShapeDtypeStruct((Bp, T, H), jnp.float32)),
        grid_spec=pltpu.PrefetchScalarGridSpec(
            num_scalar_prefetch=0,
            grid=(nchunks + 1,),
            in_specs=[
                pl.BlockSpec((Tc, Bp, H), fwd_in),
                pl.BlockSpec((Tc, Bp, H), bwd_in),
                pl.BlockSpec((Tc, Bp, H), fwd_in),
                pl.BlockSpec((Tc, Bp, H), bwd_in),
                pl.BlockSpec((2, 2 * H, 4 * H), lambda k: (0, 0, 0),
                             pipeline_mode=pl.Buffered(1)),
                pl.BlockSpec((2, H, 4 * H), lambda k: (0, 0, 0),
                             pipeline_mode=pl.Buffered(1)),
                pl.BlockSpec((2, 1, 4 * H), lambda k: (0, 0, 0),
                             pipeline_mode=pl.Buffered(1)),
            ],
            out_specs=(
                pl.BlockSpec((Bp, Tc, H), fwd_out),
                pl.BlockSpec((Bp, Tc, H), bwd_out),
            ),
            scratch_shapes=[
                pltpu.VMEM((Bp, H), jnp.float32),
                pltpu.VMEM((Bp, H), jnp.float32),
                pltpu.VMEM((Bp, H), jnp.float32),
                pltpu.VMEM((Bp, H), jnp.float32),
                pltpu.VMEM((Tc * Bp, 4 * H), jnp.bfloat16),
                pltpu.VMEM((Tc * Bp, 4 * H), jnp.bfloat16),
                pltpu.VMEM((Tc * Bp, 4 * H), jnp.bfloat16),
                pltpu.VMEM((Tc * Bp, 4 * H), jnp.bfloat16),
                pltpu.VMEM((Tc, Bp, H), jnp.float32),
                pltpu.VMEM((Tc, Bp, H), jnp.float32),
            ],
        ),
        compiler_params=pltpu.CompilerParams(
            dimension_semantics=("arbitrary",),
            vmem_limit_bytes=_VMEM_LIMIT),
    )(outf, outf, outb, outb, wih, whh, bias)


def _repack(w, axis):
    i, f, g, o = jnp.split(w, 4, axis=axis)
    return jnp.concatenate([i, f, o, g], axis=axis)


def _prep_dir(w_ih, w_hh, b_ih, b_hh):
    wih_t = _repack(w_ih, 0).T
    whh_t = _repack(w_hh, 0).T
    bias = _repack(b_ih + b_hh, 0).reshape(1, -1)
    return wih_t, whh_t, bias


def kernel(x,
           w_ih_0_0, w_hh_0_0, b_ih_0_0, b_hh_0_0,
           w_ih_0_1, w_hh_0_1, b_ih_0_1, b_hh_0_1,
           w_ih_1_0, w_hh_1_0, b_ih_1_0, b_hh_1_0,
           w_ih_1_1, w_hh_1_1, b_ih_1_1, b_hh_1_1):
    B, T, D = x.shape
    H = w_hh_0_0.shape[1]
    Bp = ((B + 7) // 8) * 8
    if Bp != B:
        x = jnp.pad(x, ((0, Bp - B), (0, 0), (0, 0)))

    w0f = _prep_dir(w_ih_0_0, w_hh_0_0, b_ih_0_0, b_hh_0_0)
    w0b = _prep_dir(w_ih_0_1, w_hh_0_1, b_ih_0_1, b_hh_0_1)
    w1f = _prep_dir(w_ih_1_0, w_hh_1_0, b_ih_1_0, b_hh_1_0)
    w1b = _prep_dir(w_ih_1_1, w_hh_1_1, b_ih_1_1, b_hh_1_1)

    wih0 = jnp.stack([w0f[0], w0b[0]]).astype(jnp.bfloat16)
    whh0 = jnp.stack([w0f[1], w0b[1]])
    b0 = jnp.stack([w0f[2], w0b[2]])
    wih1 = jnp.stack([w1f[0], w1b[0]]).astype(jnp.bfloat16)
    whh1 = jnp.stack([w1f[1], w1b[1]])
    b1 = jnp.stack([w1f[2], w1b[2]])

    Tc = 16 if T % 16 == 0 else 1
    outf0, outb0 = _run_layer0(x, wih0, whh0, b0, T, Bp, H, Tc)
    of, ob = _run_layer1(outf0, outb0, wih1, whh1, b1, T, Bp, H, Tc)
    return jnp.concatenate([of[:B], ob[:B]], axis=-1)


# speedup vs baseline: 1.0699x; 1.0699x over previous
import jax
import jax.numpy as jnp
from jax.experimental import pallas as pl
from jax.experimental.pallas import tpu as pltpu


def _lstm_gates(gates, c, H):
    sig = jax.nn.sigmoid(gates[:, :3 * H])
    i_g = sig[:, 0:H]
    f_g = sig[:, H:2 * H]
    o_g = sig[:, 2 * H:3 * H]
    g_g = jnp.tanh(gates[:, 3 * H:])
    c_new = f_g * c + i_g * g_g
    h_new = o_g * jnp.tanh(c_new)
    return h_new, c_new


def _scan_chunk(gxf_ref, gxb_ref, whh_f, whh_b, outf_ref, outb_ref,
                hf_ref, cf_ref, hb_ref, cb_ref, Tc, Bp, H):
    def body(j, carry):
        h_f, c_f, h_b, c_b = carry
        rf = pl.multiple_of(j * Bp, Bp)
        rb = pl.multiple_of((Tc - 1 - j) * Bp, Bp)
        gf = (gxf_ref[pl.ds(rf, Bp), :]
              + jnp.dot(h_f, whh_f, preferred_element_type=jnp.float32))
        gb = (gxb_ref[pl.ds(rb, Bp), :]
              + jnp.dot(h_b, whh_b, preferred_element_type=jnp.float32))
        h_f, c_f = _lstm_gates(gf, c_f, H)
        h_b, c_b = _lstm_gates(gb, c_b, H)
        outf_ref[j] = h_f.astype(outf_ref.dtype)
        outb_ref[Tc - 1 - j] = h_b.astype(outb_ref.dtype)
        return h_f, c_f, h_b, c_b

    carry = (hf_ref[...], cf_ref[...], hb_ref[...], cb_ref[...])
    carry = jax.lax.fori_loop(0, Tc, body, carry, unroll=Tc)
    h_f, c_f, h_b, c_b = carry
    hf_ref[...] = h_f
    cf_ref[...] = c_f
    hb_ref[...] = h_b
    cb_ref[...] = c_b


def _layer0_kernel(xf_ref, xb_ref, wih_ref, whh_ref, b_ref,
                   outf_ref, outb_ref,
                   hf_ref, cf_ref, hb_ref, cb_ref, gxf_ref, gxb_ref):
    k = pl.program_id(0)

    @pl.when(k == 0)
    def _():
        hf_ref[...] = jnp.zeros_like(hf_ref)
        cf_ref[...] = jnp.zeros_like(cf_ref)
        hb_ref[...] = jnp.zeros_like(hb_ref)
        cb_ref[...] = jnp.zeros_like(cb_ref)

    Bp, Tc, D = xf_ref.shape
    H = hf_ref.shape[1]

    xf2 = jnp.swapaxes(xf_ref[...], 0, 1).reshape(Tc * Bp, D)
    xb2 = jnp.swapaxes(xb_ref[...], 0, 1).reshape(Tc * Bp, D)
    gxf_ref[...] = (jnp.dot(xf2, wih_ref[0],
                            preferred_element_type=jnp.float32) + b_ref[0])
    gxb_ref[...] = (jnp.dot(xb2, wih_ref[1],
                            preferred_element_type=jnp.float32) + b_ref[1])

    _scan_chunk(gxf_ref, gxb_ref, whh_ref[0], whh_ref[1],
                outf_ref, outb_ref, hf_ref, cf_ref, hb_ref, cb_ref,
                Tc, Bp, H)


def _layer1_kernel(ff_ref, fb_ref, bf_ref, bb_ref, wih_ref, whh_ref, b_ref,
                   outf_ref, outb_ref,
                   hf_ref, cf_ref, hb_ref, cb_ref, gxf_ref, gxb_ref,
                   sf_ref, sb_ref):
    k = pl.program_id(0)

    @pl.when(k == 0)
    def _():
        hf_ref[...] = jnp.zeros_like(hf_ref)
        cf_ref[...] = jnp.zeros_like(cf_ref)
        hb_ref[...] = jnp.zeros_like(hb_ref)
        cb_ref[...] = jnp.zeros_like(cb_ref)

    Tc, Bp, H = ff_ref.shape
    wih_f = wih_ref[0]
    wih_b = wih_ref[1]

    gxf_ref[...] = (
        jnp.dot(ff_ref[...].reshape(Tc * Bp, H), wih_f[:H],
                preferred_element_type=jnp.float32)
        + jnp.dot(bf_ref[...].reshape(Tc * Bp, H), wih_f[H:],
                  preferred_element_type=jnp.float32)
        + b_ref[0])
    gxb_ref[...] = (
        jnp.dot(fb_ref[...].reshape(Tc * Bp, H), wih_b[:H],
                preferred_element_type=jnp.float32)
        + jnp.dot(bb_ref[...].reshape(Tc * Bp, H), wih_b[H:],
                  preferred_element_type=jnp.float32)
        + b_ref[1])

    _scan_chunk(gxf_ref, gxb_ref, whh_ref[0], whh_ref[1],
                sf_ref, sb_ref, hf_ref, cf_ref, hb_ref, cb_ref,
                Tc, Bp, H)

    outf_ref[...] = jnp.swapaxes(sf_ref[...], 0, 1)
    outb_ref[...] = jnp.swapaxes(sb_ref[...], 0, 1)


_VMEM_LIMIT = 100 * 1024 * 1024


def _run_layer0(x_tm, wih, whh, bias, T, Bp, H, Tc):
    D = x_tm.shape[-1]
    nchunks = T // Tc
    n1 = nchunks - 1
    return pl.pallas_call(
        _layer0_kernel,
        out_shape=(jax.ShapeDtypeStruct((T, Bp, H), jnp.bfloat16),
                   jax.ShapeDtypeStruct((T, Bp, H), jnp.bfloat16)),
        grid_spec=pltpu.PrefetchScalarGridSpec(
            num_scalar_prefetch=0,
            grid=(nchunks,),
            in_specs=[
                pl.BlockSpec((Bp, Tc, D), lambda k: (0, k, 0)),
                pl.BlockSpec((Bp, Tc, D), lambda k: (0, n1 - k, 0)),
                pl.BlockSpec((2, D, 4 * H), lambda k: (0, 0, 0),
                             pipeline_mode=pl.Buffered(1)),
                pl.BlockSpec((2, H, 4 * H), lambda k: (0, 0, 0),
                             pipeline_mode=pl.Buffered(1)),
                pl.BlockSpec((2, 1, 4 * H), lambda k: (0, 0, 0),
                             pipeline_mode=pl.Buffered(1)),
            ],
            out_specs=(
                pl.BlockSpec((Tc, Bp, H), lambda k: (k, 0, 0)),
                pl.BlockSpec((Tc, Bp, H), lambda k: (n1 - k, 0, 0)),
            ),
            scratch_shapes=[
                pltpu.VMEM((Bp, H), jnp.float32),
                pltpu.VMEM((Bp, H), jnp.float32),
                pltpu.VMEM((Bp, H), jnp.float32),
                pltpu.VMEM((Bp, H), jnp.float32),
                pltpu.VMEM((Tc * Bp, 4 * H), jnp.float32),
                pltpu.VMEM((Tc * Bp, 4 * H), jnp.float32),
            ],
        ),
        compiler_params=pltpu.CompilerParams(
            dimension_semantics=("arbitrary",),
            vmem_limit_bytes=_VMEM_LIMIT),
    )(x_tm, x_tm, wih, whh, bias)


def _run_layer1(outf, outb, wih, whh, bias, T, Bp, H, Tc):
    nchunks = T // Tc
    n1 = nchunks - 1
    return pl.pallas_call(
        _layer1_kernel,
        out_shape=(jax.ShapeDtypeStruct((Bp, T, H), jnp.float32),
                   jax.ShapeDtypeStruct((Bp, T, H), jnp.float32)),
        grid_spec=pltpu.PrefetchScalarGridSpec(
            num_scalar_prefetch=0,
            grid=(nchunks,),
            in_specs=[
                pl.BlockSpec((Tc, Bp, H), lambda k: (k, 0, 0)),
                pl.BlockSpec((Tc, Bp, H), lambda k: (n1 - k, 0, 0)),
                pl.BlockSpec((Tc, Bp, H), lambda k: (k, 0, 0)),
                pl.BlockSpec((Tc, Bp, H), lambda k: (n1 - k, 0, 0)),
                pl.BlockSpec((2, 2 * H, 4 * H), lambda k: (0, 0, 0),
                             pipeline_mode=pl.Buffered(1)),
                pl.BlockSpec((2, H, 4 * H), lambda k: (0, 0, 0),
                             pipeline_mode=pl.Buffered(1)),
                pl.BlockSpec((2, 1, 4 * H), lambda k: (0, 0, 0),
                             pipeline_mode=pl.Buffered(1)),
            ],
            out_specs=(
                pl.BlockSpec((Bp, Tc, H), lambda k: (0, k, 0)),
                pl.BlockSpec((Bp, Tc, H), lambda k: (0, n1 - k, 0)),
            ),
            scratch_shapes=[
                pltpu.VMEM((Bp, H), jnp.float32),
                pltpu.VMEM((Bp, H), jnp.float32),
                pltpu.VMEM((Bp, H), jnp.float32),
                pltpu.VMEM((Bp, H), jnp.float32),
                pltpu.VMEM((Tc * Bp, 4 * H), jnp.float32),
                pltpu.VMEM((Tc * Bp, 4 * H), jnp.float32),
                pltpu.VMEM((Tc, Bp, H), jnp.float32),
                pltpu.VMEM((Tc, Bp, H), jnp.float32),
            ],
        ),
        compiler_params=pltpu.CompilerParams(
            dimension_semantics=("arbitrary",),
            vmem_limit_bytes=_VMEM_LIMIT),
    )(outf, outf, outb, outb, wih, whh, bias)


def _repack(w, axis):
    i, f, g, o = jnp.split(w, 4, axis=axis)
    return jnp.concatenate([i, f, o, g], axis=axis)


def _prep_dir(w_ih, w_hh, b_ih, b_hh):
    wih_t = _repack(w_ih, 0).T
    whh_t = _repack(w_hh, 0).T
    bias = _repack(b_ih + b_hh, 0).reshape(1, -1)
    return wih_t, whh_t, bias


def kernel(x,
           w_ih_0_0, w_hh_0_0, b_ih_0_0, b_hh_0_0,
           w_ih_0_1, w_hh_0_1, b_ih_0_1, b_hh_0_1,
           w_ih_1_0, w_hh_1_0, b_ih_1_0, b_hh_1_0,
           w_ih_1_1, w_hh_1_1, b_ih_1_1, b_hh_1_1):
    B, T, D = x.shape
    H = w_hh_0_0.shape[1]
    Bp = ((B + 7) // 8) * 8

    if Bp != B:
        x = jnp.pad(x, ((0, Bp - B), (0, 0), (0, 0)))

    w0f = _prep_dir(w_ih_0_0, w_hh_0_0, b_ih_0_0, b_hh_0_0)
    w0b = _prep_dir(w_ih_0_1, w_hh_0_1, b_ih_0_1, b_hh_0_1)
    w1f = _prep_dir(w_ih_1_0, w_hh_1_0, b_ih_1_0, b_hh_1_0)
    w1b = _prep_dir(w_ih_1_1, w_hh_1_1, b_ih_1_1, b_hh_1_1)

    wih0 = jnp.stack([w0f[0], w0b[0]]).astype(jnp.bfloat16)
    whh0 = jnp.stack([w0f[1], w0b[1]])
    b0 = jnp.stack([w0f[2], w0b[2]])
    wih1 = jnp.stack([w1f[0], w1b[0]]).astype(jnp.bfloat16)
    whh1 = jnp.stack([w1f[1], w1b[1]])
    b1 = jnp.stack([w1f[2], w1b[2]])

    Tc = 16 if T % 16 == 0 else 1
    outf0, outb0 = _run_layer0(x, wih0, whh0, b0, T, Bp, H, Tc)
    of, ob = _run_layer1(outf0, outb0, wih1, whh1, b1, T, Bp, H, Tc)
    return jnp.concatenate([of[:B], ob[:B]], axis=-1)
